# baseline (speedup 1.0000x reference)
import functools
import math

import jax
import jax.numpy as jnp
import numpy as np
from jax import lax
from jax.experimental import pallas as pl
from jax.experimental.pallas import tpu as pltpu
from jax.experimental.pallas import tpu_sc as plsc

_NW = 32
_CHUNK = 128
_NBUF = 6


def _pe_row_table(d_model: int, n: int) -> np.ndarray:
    dm = d_model // 2
    div = np.exp(np.arange(0.0, dm, 2) * -(math.log(10000.0) / dm))
    pos = np.arange(0.0, n)[:, None] * div
    tab = np.empty((n, dm), dtype=np.float32)
    tab[:, 0::2] = np.sin(pos).astype(np.float32)
    tab[:, 1::2] = np.cos(pos).astype(np.float32)
    return tab * np.float32(0.1)


def _sc_gather_add(x, idxi, table):
    nb, nl, d = x.shape
    d2 = d // 2
    w_per_b = _NW // nb if _NW >= nb else 1
    l_per_w = nl // (_NW // nb)
    c_pos = _CHUNK // 2
    n_chunks = l_per_w // c_pos
    mesh = plsc.VectorSubcoreMesh(core_axis_name="c", subcore_axis_name="s")

    @functools.partial(
        pl.kernel,
        out_type=jax.ShapeDtypeStruct((nb, nl, d), jnp.float32),
        mesh=mesh,
        scratch_types=[
            pltpu.VMEM((n_chunks, _CHUNK), jnp.int32),
            pltpu.VMEM((_NBUF, 2, c_pos, d2), jnp.float32),
            pltpu.VMEM_SHARED(table.shape, jnp.float32),
            pltpu.SemaphoreType.DMA,
            [pltpu.SemaphoreType.DMA] * (2 * _NBUF),
            [pltpu.SemaphoreType.DMA] * _NBUF,
            [pltpu.SemaphoreType.DMA] * (2 * _NBUF),
        ],
    )
    def k(x_hbm, idx_hbm, t_hbm, out_hbm, idx_v, xb, tsh, si, sl, sg, ss):
        w = lax.axis_index("s") * 2 + lax.axis_index("c")
        bb = w // w_per_b
        c0 = (w % w_per_b) * n_chunks
        l0 = (w % w_per_b) * l_per_w
        idma = pltpu.async_copy(idx_hbm.at[bb, pl.ds(c0, n_chunks)], idx_v, si)

        loads = [None] * n_chunks
        gathers = [None] * n_chunks
        stores = [None] * n_chunks

        def start_load(j):
            b = j % _NBUF
            ls = pl.ds(l0 + j * c_pos, c_pos)
            loads[j] = [
                pltpu.async_copy(
                    x_hbm.at[bb, ls, pl.ds(h * d2, d2)],
                    xb.at[b, h], sl[2 * b + h])
                for h in range(2)
            ]

        for j in range(min(_NBUF, n_chunks)):
            start_load(j)

        @pl.when(lax.axis_index("s") == 0)
        def _():
            pltpu.sync_copy(t_hbm, tsh)

        plsc.subcore_barrier()

        idma.wait()

        for j in range(n_chunks + 2):
            if _NBUF <= j < n_chunks:
                for st in stores[j - _NBUF]:
                    st.wait()
                start_load(j)
            if 1 <= j < n_chunks + 1:
                jj = j - 1
                b = jj % _NBUF
                for ld in loads[jj]:
                    ld.wait()
                gathers[jj] = pltpu.async_copy(
                    tsh.at[idx_v.at[jj]],
                    xb.at[b].reshape(_CHUNK, d2), sg[b], add=True)
            if j >= 2:
                jj = j - 2
                b = jj % _NBUF
                gathers[jj].wait()
                ls = pl.ds(l0 + jj * c_pos, c_pos)
                stores[jj] = [
                    pltpu.async_copy(
                        xb.at[b, h], out_hbm.at[bb, ls, pl.ds(h * d2, d2)],
                        ss[2 * b + h])
                    for h in range(2)
                ]
        for j in range(max(0, n_chunks - _NBUF), n_chunks):
            for st in stores[j]:
                st.wait()

    return k(x, idxi, table)


def kernel(x, coord, pe):
    nb, nl, d = x.shape
    table = jnp.asarray(_pe_row_table(d, pe.shape[1]))
    idx = (coord / 100.0).astype(jnp.int32).reshape(nb, -1)
    k = np.arange(2 * nl)
    perm = ((k // 128) * 64 + k % 64) * 2 + (1 - (k % 128) // 64)
    idxg = jnp.take(idx, jnp.asarray(perm, jnp.int32), axis=1)
    idxg = idxg.reshape(nb, -1, 128)
    return _sc_gather_add(x, idxg, table)


# speedup vs baseline: 1.1551x; 1.1551x over previous
import functools
import math

import jax
import jax.numpy as jnp
import numpy as np
from jax import lax
from jax.experimental import pallas as pl
from jax.experimental.pallas import tpu as pltpu
from jax.experimental.pallas import tpu_sc as plsc

_NW = 32
_CHUNK = 128
_NBUF = 6


def _pe_row_table(d_model: int, n: int) -> np.ndarray:
    dm = d_model // 2
    div = np.exp(np.arange(0.0, dm, 2) * -(math.log(10000.0) / dm))
    pos = np.arange(0.0, n)[:, None] * div
    tab = np.empty((n, dm), dtype=np.float32)
    tab[:, 0::2] = np.sin(pos).astype(np.float32)
    tab[:, 1::2] = np.cos(pos).astype(np.float32)
    return tab * np.float32(0.1)


def _sc_gather_add(x, idxi, table):
    nb, nl, d = x.shape
    d2 = d // 2
    w_per_b = _NW // nb if _NW >= nb else 1
    l_per_w = nl // (_NW // nb)
    c_pos = _CHUNK // 2
    n_chunks = l_per_w // c_pos
    mesh = plsc.VectorSubcoreMesh(core_axis_name="c", subcore_axis_name="s")

    @functools.partial(
        pl.kernel,
        out_type=jax.ShapeDtypeStruct((nb, nl, d), jnp.float32),
        mesh=mesh,
        scratch_types=[
            pltpu.VMEM((n_chunks, _CHUNK), jnp.int32),
            pltpu.VMEM((_NBUF, 2, c_pos, d2), jnp.float32),
            pltpu.VMEM_SHARED(table.shape, jnp.float32),
            pltpu.SemaphoreType.DMA,
            [pltpu.SemaphoreType.DMA] * (2 * _NBUF),
            [pltpu.SemaphoreType.DMA] * _NBUF,
            [pltpu.SemaphoreType.DMA] * (2 * _NBUF),
        ],
    )
    def k(x_hbm, idx_hbm, t_hbm, out_hbm, idx_v, xb, tsh, si, sl, sg, ss):
        w = lax.axis_index("s") * 2 + lax.axis_index("c")
        bb = w // w_per_b
        c0 = (w % w_per_b) * n_chunks
        l0 = (w % w_per_b) * l_per_w
        idma = pltpu.async_copy(idx_hbm.at[bb, pl.ds(c0, n_chunks)], idx_v, si)

        loads = [None] * n_chunks
        gathers = [None] * n_chunks
        stores = [None] * n_chunks

        def start_load(j):
            b = j % _NBUF
            ls = pl.ds(l0 + j * c_pos, c_pos)
            loads[j] = [
                pltpu.async_copy(
                    x_hbm.at[bb, ls, pl.ds(h * d2, d2)],
                    xb.at[b, h], sl[2 * b + h])
                for h in range(2)
            ]

        for j in range(min(_NBUF, n_chunks)):
            start_load(j)

        @pl.when(lax.axis_index("s") == 0)
        def _():
            pltpu.sync_copy(t_hbm, tsh)

        plsc.subcore_barrier()

        idma.wait()

        for j in range(n_chunks + 2):
            if _NBUF <= j < n_chunks:
                for st in stores[j - _NBUF]:
                    st.wait()
                start_load(j)
            if 1 <= j < n_chunks + 1:
                jj = j - 1
                b = jj % _NBUF
                for ld in loads[jj]:
                    ld.wait()
                gathers[jj] = pltpu.async_copy(
                    tsh.at[idx_v.at[jj]],
                    xb.at[b].reshape(_CHUNK, d2), sg[b], add=True)
            if j >= 2:
                jj = j - 2
                b = jj % _NBUF
                gathers[jj].wait()
                ls = pl.ds(l0 + jj * c_pos, c_pos)
                stores[jj] = [
                    pltpu.async_copy(
                        xb.at[b, h], out_hbm.at[bb, ls, pl.ds(h * d2, d2)],
                        ss[2 * b + h])
                    for h in range(2)
                ]
        for j in range(max(0, n_chunks - _NBUF), n_chunks):
            for st in stores[j]:
                st.wait()

    return k(x, idxi, table)


def kernel(x, coord, pe):
    nb, nl, d = x.shape
    table = jnp.asarray(_pe_row_table(d, pe.shape[1]))
    idx = (coord / 100.0).astype(jnp.int32)
    idxg = jnp.flip(idx, -1).reshape(nb, -1, 64, 2)
    idxg = idxg.transpose(0, 1, 3, 2).reshape(nb, -1, 128)
    return _sc_gather_add(x, idxg, table)
